# Initial kernel scaffold; baseline (speedup 1.0000x reference)
import jax
import jax.numpy as jnp
from jax.experimental import pallas as pl


def kernel(pos_u, pos_v, neg_v, u_weight, v_weight):
    raise NotImplementedError("write your pallas kernel here")


# speedup vs baseline: 1.5727x; 1.5727x over previous
import functools

import jax
import jax.numpy as jnp
from jax import lax
from jax.experimental import pallas as pl
from jax.experimental.pallas import tpu as pltpu
from jax.experimental.pallas import tpu_sc as plsc

_VOCAB = 1000000
_D = 64
_B = 16384
_NEG = 5

_NW = 32
_RW = _B // _NW
_CH = 128
_NCH = _RW // _CH
_NG = _CH // 16

_LN2 = 0.6931471805599453
_SQRT2 = 1.4142135623730951


def _log_f32(y):
    bits = plsc.bitcast(y, jnp.int32)
    e = ((bits >> 23) & 0xFF) - 127
    m = plsc.bitcast((bits & 0x7FFFFF) | (127 << 23), jnp.float32)
    big = m > _SQRT2
    m = jnp.where(big, m * 0.5, m)
    e = e + big.astype(jnp.int32)
    r = (m - 1.0) / (m + 1.0)
    r2 = r * r
    p = r2 * (1.0 / 9.0) + (1.0 / 7.0)
    p = p * r2 + (1.0 / 5.0)
    p = p * r2 + (1.0 / 3.0)
    p = p * r2 + 1.0
    return e.astype(jnp.float32) * _LN2 + 2.0 * r * p


def _softplus(x):
    return _log_f32(1.0 + jnp.exp(x))


def _sc_partials(pos_u, pos_v, neg_flat, u_weight, v_weight):
    mesh = plsc.VectorSubcoreMesh(core_axis_name="c", subcore_axis_name="s")

    @functools.partial(
        pl.kernel,
        mesh=mesh,
        out_type=jax.ShapeDtypeStruct((_NW, 16), jnp.float32),
        compiler_params=pltpu.CompilerParams(
            needs_layout_passes=False, use_tc_tiling_on_sc=False),
        scratch_types=[
            pltpu.VMEM((_CH,), jnp.int32),
            pltpu.VMEM((_CH,), jnp.int32),
            pltpu.VMEM((_NEG, _CH), jnp.int32),
            pltpu.VMEM((_CH, _D), jnp.float32),
            pltpu.VMEM((_CH, _D), jnp.float32),
            pltpu.VMEM((_NEG * _CH, _D), jnp.float32),
            pltpu.VMEM((16,), jnp.float32),
            pltpu.SemaphoreType.DMA,
        ],
    )
    def k(pu_hbm, pv_hbm, nv_hbm, uw_hbm, vw_hbm, out_hbm,
          idx_u, idx_v, idx_n, u_buf, v_buf, n_buf, accv, sem):
        wid = lax.axis_index("s") * 2 + lax.axis_index("c")
        row0 = wid * _RW

        def chunk_body(ci, acc):
            base = row0 + ci * _CH
            pltpu.sync_copy(pu_hbm.at[pl.ds(base, _CH)], idx_u)
            pltpu.sync_copy(pv_hbm.at[pl.ds(base, _CH)], idx_v)
            for j in range(_NEG):
                pltpu.sync_copy(nv_hbm.at[pl.ds(base * _NEG + j * _CH, _CH)],
                                idx_n.at[j])
            cp_u = pltpu.async_copy(uw_hbm.at[idx_u], u_buf, sem)
            cp_v = pltpu.async_copy(vw_hbm.at[idx_v], v_buf, sem)
            cps = [pltpu.async_copy(vw_hbm.at[idx_n.at[j]],
                                    n_buf.at[pl.ds(j * _CH, _CH)], sem)
                   for j in range(_NEG)]
            cp_u.wait()
            cp_v.wait()
            for cp in cps:
                cp.wait()

            def group_body(g, acc):
                rows = g * 16 + lax.iota(jnp.int32, 16)
                pn = [rows * _NEG + k for k in range(_NEG)]

                def d_body(d, carry):
                    sp, s0, s1, s2, s3, s4 = carry
                    dsp = jnp.full((16,), 0, jnp.int32) + d
                    du = plsc.load_gather(u_buf, [rows, dsp])
                    dv = plsc.load_gather(v_buf, [rows, dsp])
                    sp = sp + du * dv
                    n0 = plsc.load_gather(n_buf, [pn[0], dsp])
                    s0 = s0 + du * n0
                    n1 = plsc.load_gather(n_buf, [pn[1], dsp])
                    s1 = s1 + du * n1
                    n2 = plsc.load_gather(n_buf, [pn[2], dsp])
                    s2 = s2 + du * n2
                    n3 = plsc.load_gather(n_buf, [pn[3], dsp])
                    s3 = s3 + du * n3
                    n4 = plsc.load_gather(n_buf, [pn[4], dsp])
                    s4 = s4 + du * n4
                    return (sp, s0, s1, s2, s3, s4)

                z = jnp.zeros((16,), jnp.float32)
                sp, s0, s1, s2, s3, s4 = lax.fori_loop(
                    0, _D, d_body, (z, z, z, z, z, z))
                val = _softplus(-jnp.clip(sp, -10.0, 10.0))
                for sk in (s0, s1, s2, s3, s4):
                    val = val + _softplus(jnp.clip(sk, -10.0, 10.0))
                return acc + val

            return lax.fori_loop(0, _NG, group_body, acc)

        acc = lax.fori_loop(0, _NCH, chunk_body, jnp.zeros((16,), jnp.float32))
        accv[...] = acc
        pltpu.sync_copy(accv, out_hbm.at[wid])

    return k(pos_u, pos_v, neg_flat, u_weight, v_weight)


def _finalize(partials):
    def body(p_ref, o_ref):
        o_ref[0, 0] = jnp.sum(p_ref[...]) * (1.0 / _B)

    out = pl.pallas_call(
        body,
        out_shape=jax.ShapeDtypeStruct((1, 1), jnp.float32),
        out_specs=pl.BlockSpec(memory_space=pltpu.SMEM),
    )(partials)
    return out[0, 0]


def kernel(pos_u, pos_v, neg_v, u_weight, v_weight):
    neg_flat = neg_v.reshape(_B * _NEG).astype(jnp.int32)
    partials = _sc_partials(pos_u.astype(jnp.int32), pos_v.astype(jnp.int32),
                            neg_flat, u_weight, v_weight)
    return _finalize(partials)
